# baseline (speedup 1.0000x reference)
import functools

import jax
import jax.numpy as jnp
from jax import lax
from jax.experimental import pallas as pl
from jax.experimental.pallas import tpu as pltpu
from jax.experimental.pallas import tpu_sc as plsc

PNODE_NUM = 4096
PNODE_DIM = 3
FNODE_NUM = 64
HIDDEN_DIM = 3
GCN = 128
N_NODES = 32768
HALF = 16384
N_EDGES = 262144
EPT = N_EDGES // 16
EC = 2048
ACC_ROWS = HALF + 16
GB = 128


def _segsum_body(W, src_hbm, dst_hbm, tab_hbm, out_hbm,
                 sv, dv, cpack, idx_ga, idx_sa, idx_gb, idx_sb,
                 rows_a, rows_b, acc, sem_a, sem_b):
    c = lax.axis_index("c")
    s = lax.axis_index("s")
    lo = c * HALF
    NV = W // 16

    zv = jnp.zeros((16,), jnp.float32)

    def zrow(i, _):
        rows_a[i // NV, pl.ds((i % NV) * 16, 16)] = zv
        return 0
    lax.fori_loop(0, GB * NV, zrow, 0)

    base = s * 1025
    for k in range(8):
        pltpu.sync_copy(rows_a, acc.at[pl.ds(base + k * 128, 128)])
    pltpu.sync_copy(rows_a, acc.at[pl.ds(base + 897, 128)])

    def chunk(q, off):
        pltpu.sync_copy(src_hbm.at[pl.ds(s * EPT + q * EC, EC)], sv)
        pltpu.sync_copy(dst_hbm.at[pl.ds(s * EPT + q * EC, EC)], dv)

        def filt(i, o):
            svv = sv[pl.ds(i * 16, 16)]
            loc = dv[pl.ds(i * 16, 16)] - lo
            m = (loc >= 0) & (loc < HALF)
            pos = plsc.cumsum(m.astype(jnp.int32)) + (o - 1)
            plsc.store_scatter(cpack, [pos], svv | (loc << 16), mask=m)
            return o + plsc.all_reduce_population_count(m)[0]
        return lax.fori_loop(0, EC // 16, filt, off)
    n_match = lax.fori_loop(0, EPT // EC, chunk, 0)

    gpk = jnp.full((16,), HALF << 16, jnp.int32)
    for k in range(GB // 16 + 1):
        cpack[pl.ds(n_match + k * 16, 16)] = gpk

    plsc.subcore_barrier()

    nb = (n_match + GB - 1) // GB

    def g_start(j, idx_g, idx_s, rows, sem):
        for i in range(GB // 16):
            pk = cpack[pl.ds(j * GB + i * 16, 16)]
            idx_g[pl.ds(i * 16, 16)] = pk & 0xFFFF
            idx_s[pl.ds(i * 16, 16)] = pk >> 16
        pltpu.async_copy(tab_hbm.at[idx_g], rows, sem)

    def g_wait(rows, sem):
        pltpu.make_async_copy(tab_hbm.at[pl.ds(0, GB)], rows, sem).wait()

    @pl.when(nb > 0)
    def _():
        g_start(0, idx_ga, idx_sa, rows_a, sem_a)

    def bat(t, _):
        j0 = 2 * t
        j1 = j0 + 1

        @pl.when(j1 < nb)
        def _():
            g_start(j1, idx_gb, idx_sb, rows_b, sem_b)
        g_wait(rows_a, sem_a)
        pltpu.sync_copy(rows_a, acc.at[idx_sa], add=True)

        @pl.when(j1 < nb)
        def _():
            @pl.when(j1 + 1 < nb)
            def _():
                g_start(j1 + 1, idx_ga, idx_sa, rows_a, sem_a)
            g_wait(rows_b, sem_b)
            pltpu.sync_copy(rows_b, acc.at[idx_sb], add=True)
        return 0
    lax.fori_loop(0, (nb + 1) // 2, bat, 0)

    plsc.subcore_barrier()

    pltpu.sync_copy(acc.at[pl.ds(s * 1024, 1024)],
                    out_hbm.at[pl.ds(lo + s * 1024, 1024)])


@functools.lru_cache(maxsize=None)
def _segsum_call(W):
    mesh = plsc.VectorSubcoreMesh(core_axis_name="c", subcore_axis_name="s")
    return pl.kernel(
        functools.partial(_segsum_body, W),
        mesh=mesh,
        compiler_params=pltpu.CompilerParams(needs_layout_passes=False,
                                             use_tc_tiling_on_sc=False),
        out_type=jax.ShapeDtypeStruct((N_NODES, W), jnp.float32),
        scratch_types=[
            pltpu.VMEM((EC,), jnp.int32),
            pltpu.VMEM((EC,), jnp.int32),
            pltpu.VMEM((EPT + GB + 16,), jnp.int32),
            pltpu.VMEM((GB,), jnp.int32),
            pltpu.VMEM((GB,), jnp.int32),
            pltpu.VMEM((GB,), jnp.int32),
            pltpu.VMEM((GB,), jnp.int32),
            pltpu.VMEM((GB, W), jnp.float32),
            pltpu.VMEM((GB, W), jnp.float32),
            pltpu.VMEM_SHARED((ACC_ROWS, W), jnp.float32),
            pltpu.SemaphoreType.DMA,
            pltpu.SemaphoreType.DMA,
        ],
    )


def _segsum_sc(src, dst, table):
    return _segsum_call(table.shape[1])(src, dst, table)


def _wp_matmul_body(x_ref, w_ref, b_ref, o_ref):
    o_ref[...] = jnp.dot(x_ref[...], w_ref[...],
                         preferred_element_type=jnp.float32) + b_ref[...]


def _wp_matmul(x, Wp, bp):
    TN = 256
    return pl.pallas_call(
        _wp_matmul_body,
        grid=(Wp.shape[1] // TN,),
        in_specs=[
            pl.BlockSpec((x.shape[0], x.shape[1]), lambda j: (0, 0)),
            pl.BlockSpec((Wp.shape[0], TN), lambda j: (0, j)),
            pl.BlockSpec((TN,), lambda j: (j,)),
        ],
        out_specs=pl.BlockSpec((x.shape[0], TN), lambda j: (0, j)),
        out_shape=jax.ShapeDtypeStruct((x.shape[0], Wp.shape[1]), jnp.float32),
    )(x, Wp, bp)


def _conv_body(x_ref, w1_ref, b1_ref, w2_ref, b2_ref, w3_ref, b3_ref, o_ref):
    x = x_ref[0]
    acc = jnp.zeros((64, 4089), jnp.float32)
    for k in range(8):
        acc += jnp.dot(w1_ref[k], x[:, k:k + 4089],
                       preferred_element_type=jnp.float32)
    y = jax.nn.relu(acc + b1_ref[...][:, None])
    acc = jnp.zeros((64, 4082), jnp.float32)
    for k in range(8):
        acc += jnp.dot(w2_ref[k], y[:, k:k + 4082],
                       preferred_element_type=jnp.float32)
    y = jax.nn.relu(acc + b2_ref[...][:, None])
    acc = jnp.zeros((64, 4075), jnp.float32)
    for k in range(8):
        acc += jnp.dot(w3_ref[k], y[:, k:k + 4075],
                       preferred_element_type=jnp.float32)
    o_ref[0] = jax.nn.relu(acc + b3_ref[...][:, None])


def _conv_stack(x3d, cv1_W, cv1_b, cv2_W, cv2_b, cv3_W, cv3_b):
    w1t = jnp.transpose(cv1_W, (2, 0, 1))
    w2t = jnp.transpose(cv2_W, (2, 0, 1))
    w3t = jnp.transpose(cv3_W, (2, 0, 1))
    return pl.pallas_call(
        _conv_body,
        grid=(8,),
        in_specs=[
            pl.BlockSpec((1, 128, 4096), lambda b: (b, 0, 0)),
            pl.BlockSpec((8, 64, 128), lambda b: (0, 0, 0)),
            pl.BlockSpec((64,), lambda b: (0,)),
            pl.BlockSpec((8, 64, 64), lambda b: (0, 0, 0)),
            pl.BlockSpec((64,), lambda b: (0,)),
            pl.BlockSpec((8, 64, 64), lambda b: (0, 0, 0)),
            pl.BlockSpec((64,), lambda b: (0,)),
        ],
        out_specs=pl.BlockSpec((1, 64, 4075), lambda b: (b, 0, 0)),
        out_shape=jax.ShapeDtypeStruct((8, 64, 4075), jnp.float32),
    )(x3d, w1t, cv1_b, w2t, cv2_b, w3t, cv3_b)


def _head_body(y3_ref, d1_ref, d1b_ref, d2_ref, d2b_ref, o_ref, acc_ref):
    j = pl.program_id(0)

    @pl.when(j == 0)
    def _():
        acc_ref[...] = jnp.zeros_like(acc_ref)

    part = jnp.zeros_like(acc_ref)
    for c in range(8):
        part += jnp.dot(y3_ref[:, c, :], d1_ref[pl.ds(c * 4075, 4075), :],
                        preferred_element_type=jnp.float32)
    acc_ref[...] += part

    @pl.when(j == pl.num_programs(0) - 1)
    def _():
        z = jax.nn.relu(acc_ref[...] + d1b_ref[...])
        logits = jnp.dot(z, d2_ref[...],
                         preferred_element_type=jnp.float32) + d2b_ref[...]
        mx = jnp.max(logits, axis=1, keepdims=True)
        e = jnp.exp(logits - mx)
        o_ref[...] = e / jnp.sum(e, axis=1, keepdims=True)


def _head(y3, d1_W, d1_b, d2_W, d2_b):
    return pl.pallas_call(
        _head_body,
        grid=(8,),
        in_specs=[
            pl.BlockSpec((8, 8, 4075), lambda j: (0, j, 0)),
            pl.BlockSpec((8 * 4075, 100), lambda j: (j, 0)),
            pl.BlockSpec((100,), lambda j: (0,)),
            pl.BlockSpec((100, 2), lambda j: (0, 0)),
            pl.BlockSpec((2,), lambda j: (0,)),
        ],
        out_specs=pl.BlockSpec((8, 2), lambda j: (0, 0)),
        out_shape=jax.ShapeDtypeStruct((8, 2), jnp.float32),
        scratch_shapes=[pltpu.VMEM((8, 100), jnp.float32)],
    )(y3, d1_W, d1_b, d2_W, d2_b)


def _ln(x, g, b):
    m = jnp.mean(x, axis=-1, keepdims=True)
    v = jnp.mean((x - m) ** 2, axis=-1, keepdims=True)
    return (x - m) / jnp.sqrt(v + 1e-5) * g + b


def _tables(y):
    n = y.shape[0]
    tabA = jnp.concatenate(
        [y[:, :64], jnp.ones((n, 1), jnp.float32),
         jnp.zeros((n, 15), jnp.float32)], axis=1)
    return tabA, y[:, 64:]


def _c1_body(o0_ref, xp_ref, wl_ref, bl_ref, wr_ref, g_ref, b_ref,
             tA_ref, tB_ref, yn_ref):
    o0 = o0_ref[...]
    mean = o0[:, :3] / jnp.clip(o0[:, 3:4], 1.0, None)
    y = jax.nn.relu(
        jnp.dot(mean, wl_ref[...], preferred_element_type=jnp.float32)
        + bl_ref[...]
        + jnp.dot(xp_ref[...], wr_ref[...],
                  preferred_element_type=jnp.float32))
    tA_ref[...], tB_ref[...] = _tables(y)
    yn_ref[...] = _ln(y, g_ref[...], b_ref[...])


def _c1(out0, xp0, Wl, bl, Wr, g, b):
    TT = 2048
    nv = pl.cdiv(N_NODES, TT)
    return pl.pallas_call(
        _c1_body,
        grid=(nv,),
        in_specs=[
            pl.BlockSpec((TT, 16), lambda j: (j, 0)),
            pl.BlockSpec((TT, 3), lambda j: (j, 0)),
            pl.BlockSpec((3, 128), lambda j: (0, 0)),
            pl.BlockSpec((128,), lambda j: (0,)),
            pl.BlockSpec((3, 128), lambda j: (0, 0)),
            pl.BlockSpec((128,), lambda j: (0,)),
            pl.BlockSpec((128,), lambda j: (0,)),
        ],
        out_specs=[
            pl.BlockSpec((TT, 80), lambda j: (j, 0)),
            pl.BlockSpec((TT, 64), lambda j: (j, 0)),
            pl.BlockSpec((TT, 128), lambda j: (j, 0)),
        ],
        out_shape=[
            jax.ShapeDtypeStruct((N_NODES, 80), jnp.float32),
            jax.ShapeDtypeStruct((N_NODES, 64), jnp.float32),
            jax.ShapeDtypeStruct((N_NODES, 128), jnp.float32),
        ],
    )(out0, xp0, Wl, bl, Wr, g, b)


def _c2_body(oA_ref, oB_ref, xf_ref, wl_ref, bl_ref, wr_ref, g_ref, b_ref,
             tA_ref, tB_ref):
    oA = oA_ref[...]
    cnt = jnp.clip(oA[:, 64:65], 1.0, None)
    mean = jnp.concatenate([oA[:, :64], oB_ref[...][:, :64]], axis=1) / cnt
    y = jax.nn.relu(
        jnp.dot(mean, wl_ref[...], preferred_element_type=jnp.float32)
        + bl_ref[...]
        + jnp.dot(xf_ref[...], wr_ref[...],
                  preferred_element_type=jnp.float32))
    z = _ln(y, g_ref[...], b_ref[...])
    tA_ref[...] = z[:, :64]
    tB_ref[...] = z[:, 64:]


def _c2(oA, oB, xf0, Wl, bl, Wr, g, b):
    TT = 2048
    nv = pl.cdiv(N_NODES, TT)
    return pl.pallas_call(
        _c2_body,
        grid=(nv,),
        in_specs=[
            pl.BlockSpec((TT, 80), lambda j: (j, 0)),
            pl.BlockSpec((TT, 64), lambda j: (j, 0)),
            pl.BlockSpec((TT, 3), lambda j: (j, 0)),
            pl.BlockSpec((128, 128), lambda j: (0, 0)),
            pl.BlockSpec((128,), lambda j: (0,)),
            pl.BlockSpec((3, 128), lambda j: (0, 0)),
            pl.BlockSpec((128,), lambda j: (0,)),
            pl.BlockSpec((128,), lambda j: (0,)),
        ],
        out_specs=[
            pl.BlockSpec((TT, 64), lambda j: (j, 0)),
            pl.BlockSpec((TT, 64), lambda j: (j, 0)),
        ],
        out_shape=[
            jax.ShapeDtypeStruct((N_NODES, 64), jnp.float32),
            jax.ShapeDtypeStruct((N_NODES, 64), jnp.float32),
        ],
    )(oA, oB, xf0, Wl, bl, Wr, g, b)


def _c3_body(o0_ref, oA_ref, oB_ref, xp_ref, wl_ref, bl_ref, wr_ref, y_ref):
    cnt = jnp.clip(o0_ref[...][:, 3:4], 1.0, None)
    mean = jnp.concatenate([oA_ref[...], oB_ref[...]], axis=1) / cnt
    y_ref[...] = jax.nn.relu(
        jnp.dot(mean, wl_ref[...], preferred_element_type=jnp.float32)
        + bl_ref[...]
        + jnp.dot(xp_ref[...], wr_ref[...],
                  preferred_element_type=jnp.float32))


def _c3(o0, oA, oB, xpn, Wl, bl, Wr):
    TT = 2048
    nv = pl.cdiv(N_NODES, TT)
    return pl.pallas_call(
        _c3_body,
        grid=(nv,),
        in_specs=[
            pl.BlockSpec((TT, 16), lambda j: (j, 0)),
            pl.BlockSpec((TT, 64), lambda j: (j, 0)),
            pl.BlockSpec((TT, 64), lambda j: (j, 0)),
            pl.BlockSpec((TT, 128), lambda j: (j, 0)),
            pl.BlockSpec((128, 128), lambda j: (0, 0)),
            pl.BlockSpec((128,), lambda j: (0,)),
            pl.BlockSpec((128, 128), lambda j: (0, 0)),
        ],
        out_specs=pl.BlockSpec((TT, 128), lambda j: (j, 0)),
        out_shape=jax.ShapeDtypeStruct((N_NODES, 128), jnp.float32),
    )(o0, oA, oB, xpn, Wl, bl, Wr)


def _xf_body(xs_ref, wf_ref, bf_ref, o_ref):
    o_ref[...] = jnp.dot(xs_ref[...], wf_ref[...],
                         preferred_element_type=jnp.float32) + bf_ref[...]


def _xf_matmul(x_src, Wf, bf):
    return pl.pallas_call(
        _xf_body,
        out_shape=jax.ShapeDtypeStruct((x_src.shape[0], Wf.shape[1]),
                                       jnp.float32),
    )(x_src, Wf, bf)


def kernel(x_src, x_dst, edge_index, Wp, bp, Wf, bf, f0_Wl, f0_bl, f0_Wr,
           f1_Wl, f1_bl, f1_Wr, b0_Wl, b0_bl, b0_Wr, b1_Wl, b1_bl, b1_Wr,
           ln_g, ln_b, cv1_W, cv1_b, cv2_W, cv2_b, cv3_W, cv3_b,
           d1_W, d1_b, d2_W, d2_b):
    src_f = edge_index[0, ::2]
    dst_f = edge_index[1, ::2]
    src_b = edge_index[1, 1::2]
    dst_b = edge_index[0, 1::2]

    x_p0 = _wp_matmul(jnp.reshape(x_dst, (-1, PNODE_NUM * PNODE_DIM)), Wp, bp)
    x_p0 = jnp.reshape(x_p0, (-1, HIDDEN_DIM))

    x_f0 = jnp.reshape(_xf_matmul(x_src, Wf, bf), (-1, HIDDEN_DIM))

    tab0 = jnp.concatenate(
        [x_f0, jnp.ones((N_NODES, 1), jnp.float32),
         jnp.zeros((N_NODES, 12), jnp.float32)], axis=1)
    out0 = _segsum_sc(src_f, dst_f, tab0)
    tabA1, tabB1, xp1n = _c1(out0, x_p0, f0_Wl, f0_bl, f0_Wr, ln_g, ln_b)

    outA1 = _segsum_sc(src_b, dst_b, tabA1)
    outB1 = _segsum_sc(src_b, dst_b, tabB1)
    tabA2, tabB2 = _c2(outA1, outB1, x_f0, b0_Wl, b0_bl, b0_Wr, ln_g, ln_b)

    outA2 = _segsum_sc(src_f, dst_f, tabA2)
    outB2 = _segsum_sc(src_f, dst_f, tabB2)
    x_p2 = _c3(out0, outA2, outB2, xp1n, f1_Wl, f1_bl, f1_Wr)

    x = jnp.reshape(x_p2, (-1, GCN, PNODE_NUM))
    y3 = _conv_stack(x, cv1_W, cv1_b, cv2_W, cv2_b, cv3_W, cv3_b)
    return _head(y3, d1_W, d1_b, d2_W, d2_b)


# speedup vs baseline: 1.0288x; 1.0288x over previous
import functools

import jax
import jax.numpy as jnp
from jax import lax
from jax.experimental import pallas as pl
from jax.experimental.pallas import tpu as pltpu
from jax.experimental.pallas import tpu_sc as plsc

PNODE_NUM = 4096
PNODE_DIM = 3
FNODE_NUM = 64
HIDDEN_DIM = 3
GCN = 128
N_NODES = 32768
HALF = 16384
N_EDGES = 262144
EPT = N_EDGES // 16
EC = 2048
ACC_ROWS = HALF + 16
GB = 128


def _segsum_body(W, src_hbm, dst_hbm, tab_hbm, out_hbm,
                 sva, dva, svb, dvb, cpack, idx_ga, idx_sa, idx_gb, idx_sb,
                 rows_a, rows_b, acc, sem_a, sem_b, sem_ea, sem_eb):
    c = lax.axis_index("c")
    s = lax.axis_index("s")
    lo = c * HALF
    NV = W // 16

    zv = jnp.zeros((16,), jnp.float32)

    def zrow(i, _):
        rows_a[i // NV, pl.ds((i % NV) * 16, 16)] = zv
        return 0
    lax.fori_loop(0, GB * NV, zrow, 0)

    base = s * 1025
    for k in range(8):
        pltpu.sync_copy(rows_a, acc.at[pl.ds(base + k * 128, 128)])
    pltpu.sync_copy(rows_a, acc.at[pl.ds(base + 897, 128)])

    NC = EPT // EC

    def e_start(q, svx, dvx, sem):
        pltpu.async_copy(src_hbm.at[pl.ds(s * EPT + q * EC, EC)], svx, sem)
        pltpu.async_copy(dst_hbm.at[pl.ds(s * EPT + q * EC, EC)], dvx, sem)

    def e_wait(svx, dvx, sem):
        pltpu.make_async_copy(src_hbm.at[pl.ds(0, EC)], svx, sem).wait()
        pltpu.make_async_copy(dst_hbm.at[pl.ds(0, EC)], dvx, sem).wait()

    def filt_chunk(svx, dvx, off):
        def filt(i, o):
            svv = svx[pl.ds(i * 16, 16)]
            loc = dvx[pl.ds(i * 16, 16)] - lo
            m = (loc >= 0) & (loc < HALF)
            pos = plsc.cumsum(m.astype(jnp.int32)) + (o - 1)
            plsc.store_scatter(cpack, [pos], svv | (loc << 16), mask=m)
            return o + plsc.all_reduce_population_count(m)[0]
        return lax.fori_loop(0, EC // 16, filt, off)

    e_start(0, sva, dva, sem_ea)

    def chunk2(t, off):
        q1 = 2 * t + 1
        e_start(q1, svb, dvb, sem_eb)
        e_wait(sva, dva, sem_ea)
        off = filt_chunk(sva, dva, off)

        @pl.when(q1 + 1 < NC)
        def _():
            e_start(q1 + 1, sva, dva, sem_ea)
        e_wait(svb, dvb, sem_eb)
        return filt_chunk(svb, dvb, off)
    n_match = lax.fori_loop(0, NC // 2, chunk2, 0)

    gpk = jnp.full((16,), HALF << 16, jnp.int32)
    for k in range(GB // 16 + 1):
        cpack[pl.ds(n_match + k * 16, 16)] = gpk

    plsc.subcore_barrier()

    nb = (n_match + GB - 1) // GB

    def g_start(j, idx_g, idx_s, rows, sem):
        for i in range(GB // 16):
            pk = cpack[pl.ds(j * GB + i * 16, 16)]
            idx_g[pl.ds(i * 16, 16)] = pk & 0xFFFF
            idx_s[pl.ds(i * 16, 16)] = pk >> 16
        pltpu.async_copy(tab_hbm.at[idx_g], rows, sem)

    def g_wait(rows, sem):
        pltpu.make_async_copy(tab_hbm.at[pl.ds(0, GB)], rows, sem).wait()

    @pl.when(nb > 0)
    def _():
        g_start(0, idx_ga, idx_sa, rows_a, sem_a)

    def bat(t, _):
        j0 = 2 * t
        j1 = j0 + 1

        @pl.when(j1 < nb)
        def _():
            g_start(j1, idx_gb, idx_sb, rows_b, sem_b)
        g_wait(rows_a, sem_a)
        pltpu.sync_copy(rows_a, acc.at[idx_sa], add=True)

        @pl.when(j1 < nb)
        def _():
            @pl.when(j1 + 1 < nb)
            def _():
                g_start(j1 + 1, idx_ga, idx_sa, rows_a, sem_a)
            g_wait(rows_b, sem_b)
            pltpu.sync_copy(rows_b, acc.at[idx_sb], add=True)
        return 0
    lax.fori_loop(0, (nb + 1) // 2, bat, 0)

    plsc.subcore_barrier()

    pltpu.sync_copy(acc.at[pl.ds(s * 1024, 1024)],
                    out_hbm.at[pl.ds(lo + s * 1024, 1024)])


@functools.lru_cache(maxsize=None)
def _segsum_call(W):
    mesh = plsc.VectorSubcoreMesh(core_axis_name="c", subcore_axis_name="s")
    return pl.kernel(
        functools.partial(_segsum_body, W),
        mesh=mesh,
        compiler_params=pltpu.CompilerParams(needs_layout_passes=False,
                                             use_tc_tiling_on_sc=False),
        out_type=jax.ShapeDtypeStruct((N_NODES, W), jnp.float32),
        scratch_types=[
            pltpu.VMEM((EC,), jnp.int32),
            pltpu.VMEM((EC,), jnp.int32),
            pltpu.VMEM((EC,), jnp.int32),
            pltpu.VMEM((EC,), jnp.int32),
            pltpu.VMEM((EPT + GB + 16,), jnp.int32),
            pltpu.VMEM((GB,), jnp.int32),
            pltpu.VMEM((GB,), jnp.int32),
            pltpu.VMEM((GB,), jnp.int32),
            pltpu.VMEM((GB,), jnp.int32),
            pltpu.VMEM((GB, W), jnp.float32),
            pltpu.VMEM((GB, W), jnp.float32),
            pltpu.VMEM_SHARED((ACC_ROWS, W), jnp.float32),
            pltpu.SemaphoreType.DMA,
            pltpu.SemaphoreType.DMA,
            pltpu.SemaphoreType.DMA,
            pltpu.SemaphoreType.DMA,
        ],
    )


def _segsum_sc(src, dst, table):
    return _segsum_call(table.shape[1])(src, dst, table)


def _wp_matmul_body(x_ref, w_ref, b_ref, o_ref):
    o_ref[...] = jnp.dot(x_ref[...], w_ref[...],
                         preferred_element_type=jnp.float32) + b_ref[...]


def _wp_matmul(x, Wp, bp):
    TN = 256
    return pl.pallas_call(
        _wp_matmul_body,
        grid=(Wp.shape[1] // TN,),
        in_specs=[
            pl.BlockSpec((x.shape[0], x.shape[1]), lambda j: (0, 0)),
            pl.BlockSpec((Wp.shape[0], TN), lambda j: (0, j)),
            pl.BlockSpec((TN,), lambda j: (j,)),
        ],
        out_specs=pl.BlockSpec((x.shape[0], TN), lambda j: (0, j)),
        out_shape=jax.ShapeDtypeStruct((x.shape[0], Wp.shape[1]), jnp.float32),
    )(x, Wp, bp)


def _conv_body(x_ref, w1_ref, b1_ref, w2_ref, b2_ref, w3_ref, b3_ref, o_ref):
    x = x_ref[0]
    acc = jnp.zeros((64, 4089), jnp.float32)
    for k in range(8):
        acc += jnp.dot(w1_ref[k], x[:, k:k + 4089],
                       preferred_element_type=jnp.float32)
    y = jax.nn.relu(acc + b1_ref[...][:, None])
    acc = jnp.zeros((64, 4082), jnp.float32)
    for k in range(8):
        acc += jnp.dot(w2_ref[k], y[:, k:k + 4082],
                       preferred_element_type=jnp.float32)
    y = jax.nn.relu(acc + b2_ref[...][:, None])
    acc = jnp.zeros((64, 4075), jnp.float32)
    for k in range(8):
        acc += jnp.dot(w3_ref[k], y[:, k:k + 4075],
                       preferred_element_type=jnp.float32)
    o_ref[0] = jax.nn.relu(acc + b3_ref[...][:, None])


def _conv_stack(x3d, cv1_W, cv1_b, cv2_W, cv2_b, cv3_W, cv3_b):
    w1t = jnp.transpose(cv1_W, (2, 0, 1))
    w2t = jnp.transpose(cv2_W, (2, 0, 1))
    w3t = jnp.transpose(cv3_W, (2, 0, 1))
    return pl.pallas_call(
        _conv_body,
        grid=(8,),
        in_specs=[
            pl.BlockSpec((1, 128, 4096), lambda b: (b, 0, 0)),
            pl.BlockSpec((8, 64, 128), lambda b: (0, 0, 0)),
            pl.BlockSpec((64,), lambda b: (0,)),
            pl.BlockSpec((8, 64, 64), lambda b: (0, 0, 0)),
            pl.BlockSpec((64,), lambda b: (0,)),
            pl.BlockSpec((8, 64, 64), lambda b: (0, 0, 0)),
            pl.BlockSpec((64,), lambda b: (0,)),
        ],
        out_specs=pl.BlockSpec((1, 64, 4075), lambda b: (b, 0, 0)),
        out_shape=jax.ShapeDtypeStruct((8, 64, 4075), jnp.float32),
    )(x3d, w1t, cv1_b, w2t, cv2_b, w3t, cv3_b)


def _head_body(y3_ref, d1_ref, d1b_ref, d2_ref, d2b_ref, o_ref, acc_ref):
    j = pl.program_id(0)

    @pl.when(j == 0)
    def _():
        acc_ref[...] = jnp.zeros_like(acc_ref)

    part = jnp.zeros_like(acc_ref)
    for c in range(8):
        part += jnp.dot(y3_ref[:, c, :], d1_ref[pl.ds(c * 4075, 4075), :],
                        preferred_element_type=jnp.float32)
    acc_ref[...] += part

    @pl.when(j == pl.num_programs(0) - 1)
    def _():
        z = jax.nn.relu(acc_ref[...] + d1b_ref[...])
        logits = jnp.dot(z, d2_ref[...],
                         preferred_element_type=jnp.float32) + d2b_ref[...]
        mx = jnp.max(logits, axis=1, keepdims=True)
        e = jnp.exp(logits - mx)
        o_ref[...] = e / jnp.sum(e, axis=1, keepdims=True)


def _head(y3, d1_W, d1_b, d2_W, d2_b):
    return pl.pallas_call(
        _head_body,
        grid=(8,),
        in_specs=[
            pl.BlockSpec((8, 8, 4075), lambda j: (0, j, 0)),
            pl.BlockSpec((8 * 4075, 100), lambda j: (j, 0)),
            pl.BlockSpec((100,), lambda j: (0,)),
            pl.BlockSpec((100, 2), lambda j: (0, 0)),
            pl.BlockSpec((2,), lambda j: (0,)),
        ],
        out_specs=pl.BlockSpec((8, 2), lambda j: (0, 0)),
        out_shape=jax.ShapeDtypeStruct((8, 2), jnp.float32),
        scratch_shapes=[pltpu.VMEM((8, 100), jnp.float32)],
    )(y3, d1_W, d1_b, d2_W, d2_b)


def _ln(x, g, b):
    m = jnp.mean(x, axis=-1, keepdims=True)
    v = jnp.mean((x - m) ** 2, axis=-1, keepdims=True)
    return (x - m) / jnp.sqrt(v + 1e-5) * g + b


def _tables(y):
    n = y.shape[0]
    tabA = jnp.concatenate(
        [y[:, :64], jnp.ones((n, 1), jnp.float32),
         jnp.zeros((n, 15), jnp.float32)], axis=1)
    return tabA, y[:, 64:]


def _c1_body(o0_ref, xp_ref, wl_ref, bl_ref, wr_ref, g_ref, b_ref,
             tA_ref, tB_ref, yn_ref):
    o0 = o0_ref[...]
    mean = o0[:, :3] / jnp.clip(o0[:, 3:4], 1.0, None)
    y = jax.nn.relu(
        jnp.dot(mean, wl_ref[...], preferred_element_type=jnp.float32)
        + bl_ref[...]
        + jnp.dot(xp_ref[...], wr_ref[...],
                  preferred_element_type=jnp.float32))
    tA_ref[...], tB_ref[...] = _tables(y)
    yn_ref[...] = _ln(y, g_ref[...], b_ref[...])


def _c1(out0, xp0, Wl, bl, Wr, g, b):
    TT = 2048
    nv = pl.cdiv(N_NODES, TT)
    return pl.pallas_call(
        _c1_body,
        grid=(nv,),
        in_specs=[
            pl.BlockSpec((TT, 16), lambda j: (j, 0)),
            pl.BlockSpec((TT, 3), lambda j: (j, 0)),
            pl.BlockSpec((3, 128), lambda j: (0, 0)),
            pl.BlockSpec((128,), lambda j: (0,)),
            pl.BlockSpec((3, 128), lambda j: (0, 0)),
            pl.BlockSpec((128,), lambda j: (0,)),
            pl.BlockSpec((128,), lambda j: (0,)),
        ],
        out_specs=[
            pl.BlockSpec((TT, 80), lambda j: (j, 0)),
            pl.BlockSpec((TT, 64), lambda j: (j, 0)),
            pl.BlockSpec((TT, 128), lambda j: (j, 0)),
        ],
        out_shape=[
            jax.ShapeDtypeStruct((N_NODES, 80), jnp.float32),
            jax.ShapeDtypeStruct((N_NODES, 64), jnp.float32),
            jax.ShapeDtypeStruct((N_NODES, 128), jnp.float32),
        ],
    )(out0, xp0, Wl, bl, Wr, g, b)


def _c2_body(oA_ref, oB_ref, xf_ref, wl_ref, bl_ref, wr_ref, g_ref, b_ref,
             tA_ref, tB_ref):
    oA = oA_ref[...]
    cnt = jnp.clip(oA[:, 64:65], 1.0, None)
    mean = jnp.concatenate([oA[:, :64], oB_ref[...][:, :64]], axis=1) / cnt
    y = jax.nn.relu(
        jnp.dot(mean, wl_ref[...], preferred_element_type=jnp.float32)
        + bl_ref[...]
        + jnp.dot(xf_ref[...], wr_ref[...],
                  preferred_element_type=jnp.float32))
    z = _ln(y, g_ref[...], b_ref[...])
    tA_ref[...] = z[:, :64]
    tB_ref[...] = z[:, 64:]


def _c2(oA, oB, xf0, Wl, bl, Wr, g, b):
    TT = 2048
    nv = pl.cdiv(N_NODES, TT)
    return pl.pallas_call(
        _c2_body,
        grid=(nv,),
        in_specs=[
            pl.BlockSpec((TT, 80), lambda j: (j, 0)),
            pl.BlockSpec((TT, 64), lambda j: (j, 0)),
            pl.BlockSpec((TT, 3), lambda j: (j, 0)),
            pl.BlockSpec((128, 128), lambda j: (0, 0)),
            pl.BlockSpec((128,), lambda j: (0,)),
            pl.BlockSpec((3, 128), lambda j: (0, 0)),
            pl.BlockSpec((128,), lambda j: (0,)),
            pl.BlockSpec((128,), lambda j: (0,)),
        ],
        out_specs=[
            pl.BlockSpec((TT, 64), lambda j: (j, 0)),
            pl.BlockSpec((TT, 64), lambda j: (j, 0)),
        ],
        out_shape=[
            jax.ShapeDtypeStruct((N_NODES, 64), jnp.float32),
            jax.ShapeDtypeStruct((N_NODES, 64), jnp.float32),
        ],
    )(oA, oB, xf0, Wl, bl, Wr, g, b)


def _c3_body(o0_ref, oA_ref, oB_ref, xp_ref, wl_ref, bl_ref, wr_ref, y_ref):
    cnt = jnp.clip(o0_ref[...][:, 3:4], 1.0, None)
    mean = jnp.concatenate([oA_ref[...], oB_ref[...]], axis=1) / cnt
    y_ref[...] = jax.nn.relu(
        jnp.dot(mean, wl_ref[...], preferred_element_type=jnp.float32)
        + bl_ref[...]
        + jnp.dot(xp_ref[...], wr_ref[...],
                  preferred_element_type=jnp.float32))


def _c3(o0, oA, oB, xpn, Wl, bl, Wr):
    TT = 2048
    nv = pl.cdiv(N_NODES, TT)
    return pl.pallas_call(
        _c3_body,
        grid=(nv,),
        in_specs=[
            pl.BlockSpec((TT, 16), lambda j: (j, 0)),
            pl.BlockSpec((TT, 64), lambda j: (j, 0)),
            pl.BlockSpec((TT, 64), lambda j: (j, 0)),
            pl.BlockSpec((TT, 128), lambda j: (j, 0)),
            pl.BlockSpec((128, 128), lambda j: (0, 0)),
            pl.BlockSpec((128,), lambda j: (0,)),
            pl.BlockSpec((128, 128), lambda j: (0, 0)),
        ],
        out_specs=pl.BlockSpec((TT, 128), lambda j: (j, 0)),
        out_shape=jax.ShapeDtypeStruct((N_NODES, 128), jnp.float32),
    )(o0, oA, oB, xpn, Wl, bl, Wr)


def _xf_body(xs_ref, wf_ref, bf_ref, o_ref):
    o_ref[...] = jnp.dot(xs_ref[...], wf_ref[...],
                         preferred_element_type=jnp.float32) + bf_ref[...]


def _xf_matmul(x_src, Wf, bf):
    return pl.pallas_call(
        _xf_body,
        out_shape=jax.ShapeDtypeStruct((x_src.shape[0], Wf.shape[1]),
                                       jnp.float32),
    )(x_src, Wf, bf)


def kernel(x_src, x_dst, edge_index, Wp, bp, Wf, bf, f0_Wl, f0_bl, f0_Wr,
           f1_Wl, f1_bl, f1_Wr, b0_Wl, b0_bl, b0_Wr, b1_Wl, b1_bl, b1_Wr,
           ln_g, ln_b, cv1_W, cv1_b, cv2_W, cv2_b, cv3_W, cv3_b,
           d1_W, d1_b, d2_W, d2_b):
    src_f = edge_index[0, ::2]
    dst_f = edge_index[1, ::2]
    src_b = edge_index[1, 1::2]
    dst_b = edge_index[0, 1::2]

    x_p0 = _wp_matmul(jnp.reshape(x_dst, (-1, PNODE_NUM * PNODE_DIM)), Wp, bp)
    x_p0 = jnp.reshape(x_p0, (-1, HIDDEN_DIM))

    x_f0 = jnp.reshape(_xf_matmul(x_src, Wf, bf), (-1, HIDDEN_DIM))

    tab0 = jnp.concatenate(
        [x_f0, jnp.ones((N_NODES, 1), jnp.float32),
         jnp.zeros((N_NODES, 12), jnp.float32)], axis=1)
    out0 = _segsum_sc(src_f, dst_f, tab0)
    tabA1, tabB1, xp1n = _c1(out0, x_p0, f0_Wl, f0_bl, f0_Wr, ln_g, ln_b)

    outA1 = _segsum_sc(src_b, dst_b, tabA1)
    outB1 = _segsum_sc(src_b, dst_b, tabB1)
    tabA2, tabB2 = _c2(outA1, outB1, x_f0, b0_Wl, b0_bl, b0_Wr, ln_g, ln_b)

    outA2 = _segsum_sc(src_f, dst_f, tabA2)
    outB2 = _segsum_sc(src_f, dst_f, tabB2)
    x_p2 = _c3(out0, outA2, outB2, xp1n, f1_Wl, f1_bl, f1_Wr)

    x = jnp.reshape(x_p2, (-1, GCN, PNODE_NUM))
    y3 = _conv_stack(x, cv1_W, cv1_b, cv2_W, cv2_b, cv3_W, cv3_b)
    return _head(y3, d1_W, d1_b, d2_W, d2_b)
